# baseline (speedup 1.0000x reference)
import functools

import jax
import jax.numpy as jnp
from jax import lax
from jax.experimental import pallas as pl
from jax.experimental.pallas import tpu as pltpu
from jax.experimental.pallas import tpu_sc as plsc

_NC, _NS = 2, 16
_NW = _NC * _NS
_L = 16


def _sc_gather(tables_flat, x, f, v, d):
    bsz = x.shape[0]
    n_rows = bsz * f
    rpw = n_rows // _NW
    samples_pw = bsz // _NW
    chunk_s = 128
    chunk = chunk_s * f
    n_chunks = samples_pw // chunk_s

    mesh = plsc.VectorSubcoreMesh(core_axis_name="c", subcore_axis_name="s")

    def body(tab_hbm, x_hbm, out_hbm, x_v, idx_v, gat_v, sem):
        wid = lax.axis_index("s") * _NC + lax.axis_index("c")

        def step(i, carry):
            s0 = wid * samples_pw + i * chunk_s
            off = s0 * f
            pltpu.sync_copy(x_hbm.at[pl.ds(s0, chunk_s)], x_v)

            def build(j, rc):
                r, c = rc
                xv = plsc.load_gather(x_v, [r, c])
                idx_v[pl.ds(j * _L, _L)] = xv + c * v
                c2 = c + _L
                over = c2 >= f
                c2 = jnp.where(over, c2 - f, c2)
                r2 = jnp.where(over, r + 1, r)
                return (r2, c2)

            iot = lax.iota(jnp.int32, _L)
            lax.fori_loop(0, chunk // _L, build,
                          (jnp.zeros((_L,), jnp.int32), iot))
            pltpu.async_copy(tab_hbm.at[idx_v], gat_v, sem).wait()
            pltpu.sync_copy(gat_v, out_hbm.at[pl.ds(off, chunk)])
            return carry

        lax.fori_loop(0, n_chunks, step, 0)

    fn = pl.kernel(
        body,
        out_type=jax.ShapeDtypeStruct((n_rows, d), jnp.float32),
        mesh=mesh,
        scratch_types=[
            pltpu.VMEM((chunk_s, f), jnp.int32),
            pltpu.VMEM((chunk,), jnp.int32),
            pltpu.VMEM((chunk, d), jnp.float32),
            pltpu.SemaphoreType.DMA,
        ],
        compiler_params=pltpu.CompilerParams(
            use_tc_tiling_on_sc=False, needs_layout_passes=False
        ),
    )
    return fn(tables_flat, x)


def _tc_project(concat, wt, b):
    bsz, k = concat.shape
    o = wt.shape[1]
    bm = 2048

    def body(a_ref, w_ref, b_ref, o_ref):
        o_ref[...] = (
            jnp.dot(a_ref[...], w_ref[...], preferred_element_type=jnp.float32)
            + b_ref[...]
        )

    return pl.pallas_call(
        body,
        grid=(bsz // bm,),
        in_specs=[
            pl.BlockSpec((bm, k), lambda i: (i, 0)),
            pl.BlockSpec((k, o), lambda i: (0, 0)),
            pl.BlockSpec((1, o), lambda i: (0, 0)),
        ],
        out_specs=pl.BlockSpec((bm, o), lambda i: (i, 0)),
        out_shape=jax.ShapeDtypeStruct((bsz, o), jnp.float32),
    )(concat, wt, b.reshape(1, o))


def kernel(x, tables, W, b):
    bsz, f = x.shape
    _, v, d = tables.shape
    tables_flat = tables.reshape(f * v, d)
    concat = _sc_gather(tables_flat, x, f, v, d)
    return _tc_project(concat.reshape(bsz, f * d), W.T, b)
